# baseline (speedup 1.0000x reference)
import functools

import jax
import jax.numpy as jnp
from jax import lax
from jax.experimental import pallas as pl
from jax.experimental.pallas import tpu as pltpu
from jax.experimental.pallas import tpu_sc as plsc

N = 10000
E = 320000
D = 128
DE = 16
H = 8
F = 16
ALPHA = 0.2

NP = 10240
NW = 32
C = 104
NCH0 = 115
NCH1 = 80
EPT0 = NCH0 * C
EPT1 = NCH1 * C
EP = 16 * (EPT0 + EPT1)
RPT = NP // 16


def _mm(a, b, blk, pack=1):
    M, K = a.shape
    Nc = b.shape[1]

    def body(a_ref, b_ref, o_ref):
        av = a_ref[...]
        if pack > 1:
            av = av.reshape(blk // pack, K * pack)
        o_ref[...] = jnp.dot(av, b_ref[...],
                             preferred_element_type=jnp.float32)

    return pl.pallas_call(
        body,
        grid=(M // blk,),
        in_specs=[pl.BlockSpec((blk, K), lambda i: (i, 0)),
                  pl.BlockSpec((K * pack, Nc), lambda i: (0, 0))],
        out_specs=pl.BlockSpec((blk // pack, Nc), lambda i: (i, 0)),
        out_shape=jax.ShapeDtypeStruct((M // pack, Nc), jnp.float32),
    )(a, b)


def _h_sd(xp, W, BigBsd, blk):
    def body(a_ref, w_ref, b_ref, h_ref, sd_ref):
        av = a_ref[...]
        h_ref[...] = jnp.dot(av, w_ref[...], preferred_element_type=jnp.float32)
        av8 = av.reshape(blk // 8, 8 * D)
        sd_ref[...] = jnp.dot(av8, b_ref[...],
                              preferred_element_type=jnp.float32)

    return pl.pallas_call(
        body,
        grid=(NP // blk,),
        in_specs=[pl.BlockSpec((blk, D), lambda i: (i, 0)),
                  pl.BlockSpec((D, D), lambda i: (0, 0)),
                  pl.BlockSpec((8 * D, D), lambda i: (0, 0))],
        out_specs=[pl.BlockSpec((blk, D), lambda i: (i, 0)),
                   pl.BlockSpec((blk // 8, D), lambda i: (i, 0))],
        out_shape=[jax.ShapeDtypeStruct((NP, D), jnp.float32),
                   jax.ShapeDtypeStruct((NP // 8, D), jnp.float32)],
    )(xp, W, BigBsd)


def _le_mm(ea, Be, blk, out_rows):
    E_ = ea.shape[0]
    ea3 = ea.reshape(E_ // 8, 8, 16)

    def body(a_ref, b_ref, o_ref):
        av = a_ref[...]
        le = lax.dot_general(av, b_ref[...], (((2,), (0,)), ((), ())),
                             preferred_element_type=jnp.float32)
        o_ref[...] = le.reshape(blk, 128)

    return pl.pallas_call(
        body,
        grid=(E_ // 8 // blk,),
        in_specs=[pl.BlockSpec((blk, 8, 16), lambda i: (i, 0, 0)),
                  pl.BlockSpec((16, 16), lambda i: (0, 0))],
        out_specs=pl.BlockSpec((blk, 128), lambda i: (i, 0)),
        out_shape=jax.ShapeDtypeStruct((out_rows, 128), jnp.float32),
    )(ea3, Be)


def _combine(p_ref, d_ref, ex_ref):
    o = p_ref[0] + p_ref[1]
    r = 1.0 / (d_ref[0] + d_ref[1] + 1e-16)
    rx = jnp.dot(r, ex_ref[...], preferred_element_type=jnp.float32)
    return o * rx


def _mid(P, DEN, Ex, W2, Bsd2):
    blk = 1024

    def body(p_ref, d_ref, ex_ref, w_ref, b_ref, h2_ref, sd_ref):
        o = _combine(p_ref, d_ref, ex_ref)
        hp = jnp.where(o > 0, o, jnp.exp(o) - 1.0)
        h2_ref[...] = jnp.dot(hp, w_ref[...], preferred_element_type=jnp.float32)
        hp8 = hp.reshape(blk // 8, 8 * D)
        sd_ref[...] = jnp.dot(hp8, b_ref[...], preferred_element_type=jnp.float32)

    return pl.pallas_call(
        body,
        grid=(NP // blk,),
        in_specs=[pl.BlockSpec((2, blk, D), lambda i: (0, i, 0)),
                  pl.BlockSpec((2, blk, 16), lambda i: (0, i, 0)),
                  pl.BlockSpec((16, D), lambda i: (0, 0)),
                  pl.BlockSpec((D, D), lambda i: (0, 0)),
                  pl.BlockSpec((8 * D, D), lambda i: (0, 0))],
        out_specs=[pl.BlockSpec((blk, D), lambda i: (i, 0)),
                   pl.BlockSpec((blk // 8, D), lambda i: (i, 0))],
        out_shape=[jax.ShapeDtypeStruct((NP, D), jnp.float32),
                   jax.ShapeDtypeStruct((NP // 8, D), jnp.float32)],
    )(P, DEN, Ex, W2, Bsd2)


def _post(P, DEN, Ex):
    blk = 1024

    def body(p_ref, d_ref, ex_ref, o_ref):
        o = _combine(p_ref, d_ref, ex_ref)
        acc = o[:, 0:16]
        for hh in range(1, H):
            acc = acc + o[:, hh * 16:(hh + 1) * 16]
        o_ref[...] = acc * (1.0 / H)

    return pl.pallas_call(
        body,
        grid=(NP // blk,),
        in_specs=[pl.BlockSpec((2, blk, D), lambda i: (0, i, 0)),
                  pl.BlockSpec((2, blk, 16), lambda i: (0, i, 0)),
                  pl.BlockSpec((16, D), lambda i: (0, 0))],
        out_specs=pl.BlockSpec((blk, 16), lambda i: (i, 0)),
        out_shape=jax.ShapeDtypeStruct((NP, 16), jnp.float32),
    )(P, DEN, Ex)


_DNUMS = lax.GatherDimensionNumbers(
    offset_dims=(), collapsed_slice_dims=(0,), start_index_map=(0,))


def _dg(v, idx):
    return lax.gather(v, idx[:, None], _DNUMS, (1,),
                      mode=lax.GatherScatterMode.PROMISE_IN_BOUNDS)


def _edge_layer(srcp, dstp, SD, LE, hmat, le_hi):
    mesh = plsc.VectorSubcoreMesh(core_axis_name="c", subcore_axis_name="s")

    @functools.partial(
        pl.kernel,
        out_type=(jax.ShapeDtypeStruct((2, NP, D), jnp.float32),
                  jax.ShapeDtypeStruct((2, NP, 16), jnp.float32)),
        mesh=mesh,
        compiler_params=pltpu.CompilerParams(use_tc_tiling_on_sc=False),
        scratch_types=[
            [pltpu.VMEM((C,), jnp.int32) for _ in range(2)],
            [pltpu.VMEM((C,), jnp.int32) for _ in range(2)],
            [pltpu.VMEM((C, 16), jnp.float32) for _ in range(2)],
            [pltpu.VMEM((C, 16), jnp.float32) for _ in range(2)],
            [pltpu.VMEM((C, 16), jnp.float32) for _ in range(2)],
            [pltpu.VMEM((C, D), jnp.float32) for _ in range(2)],
            pltpu.VMEM((C, 16), jnp.float32),
            pltpu.VMEM_SHARED((NP, D), jnp.float32),
            pltpu.VMEM_SHARED((NP, 16), jnp.float32),
            [pltpu.SemaphoreType.DMA for _ in range(2)],
            [pltpu.SemaphoreType.DMA for _ in range(2)],
        ],
    )
    def k(src_hbm, dst_hbm, sd_hbm, le_hbm, h_hbm, out_hbm, den_hbm,
          idxs, idxd, sds, sdd, leb, rows, exb, out_sh, den_sh, sems, semh):
        cc = lax.axis_index("c")
        ss = lax.axis_index("s")
        ii = lax.iota(jnp.int32, 16)
        shift8 = (ii & 7) + 8
        zero16 = jnp.zeros((16,), jnp.float32)
        base_r = ss * RPT
        ebase = jnp.where(cc == 0, ss * EPT0, 16 * EPT0 + ss * EPT1)
        nch = jnp.where(cc == 0, NCH0, NCH1)

        @plsc.parallel_loop(0, C, unroll=4)
        def zrow(i):
            exb[i, :] = zero16
            for j in range(D // 16):
                rows[0][i, pl.ds(j * 16, 16)] = zero16
        for kk in range(RPT // C):
            pltpu.sync_copy(exb, den_sh.at[pl.ds(base_r + kk * C, C), :])
            pltpu.sync_copy(rows[0], out_sh.at[pl.ds(base_r + kk * C, C), :])
        rem = RPT % C
        if rem:
            r0 = base_r + (RPT // C) * C
            pltpu.sync_copy(exb.at[pl.ds(0, rem), :],
                            den_sh.at[pl.ds(r0, rem), :])
            pltpu.sync_copy(rows[0].at[pl.ds(0, rem), :],
                            out_sh.at[pl.ds(r0, rem), :])
        plsc.subcore_barrier()

        def prep(g, b):
            base = ebase + g * C
            pltpu.sync_copy(src_hbm.at[pl.ds(base, C)], idxs[b])
            pltpu.sync_copy(dst_hbm.at[pl.ds(base, C)], idxd[b])
            pltpu.async_copy(sd_hbm.at[idxs[b]], sds[b], sems[b])
            pltpu.async_copy(sd_hbm.at[idxd[b]], sdd[b], sems[b])
            pltpu.async_copy(le_hbm.at[pl.ds(base, C), :], leb[b], sems[b])

        def process(g0, b):
            @pl.when(g0 + 1 < nch)
            def _():
                pltpu.async_copy(h_hbm.at[idxs[1 - b]], rows[1 - b],
                                 semh[1 - b])

            pltpu.make_async_copy(sd_hbm.at[idxs[b]], sds[b], sems[b]).wait()
            pltpu.make_async_copy(sd_hbm.at[idxd[b]], sdd[b], sems[b]).wait()
            pltpu.make_async_copy(le_hbm.at[pl.ds(0, C), :], leb[b],
                                  sems[b]).wait()

            @plsc.parallel_loop(0, C, unroll=4)
            def edge_ex(e):
                vs = sds[b][e, :]
                vd = _dg(sdd[b][e, :], shift8)
                if le_hi:
                    vl = _dg(leb[b][e, :], shift8)
                else:
                    vl = leb[b][e, :]
                z = vs + vd + vl
                z = jnp.where(z > 0, z, z * ALPHA)
                exb[e, :] = jnp.exp(z)

            pltpu.sync_copy(exb, den_sh.at[idxd[b]], add=True)
            pltpu.make_async_copy(h_hbm.at[idxs[b]], rows[b], semh[b]).wait()

            @plsc.parallel_loop(0, C, unroll=4)
            def edge_scale(e):
                a = exb[e, :]
                for hh in range(H):
                    fh = jnp.full((16,), hh, jnp.int32)
                    sc = _dg(a, fh)
                    rows[b][e, pl.ds(hh * 16, 16)] = (
                        rows[b][e, pl.ds(hh * 16, 16)] * sc)

            pltpu.sync_copy(rows[b], out_sh.at[idxd[b]], add=True)

            @pl.when(g0 + 2 < nch)
            def _():
                prep(g0 + 2, b)

        prep(0, 0)
        pltpu.async_copy(h_hbm.at[idxs[0]], rows[0], semh[0])
        prep(1, 1)

        def pair(kk, carry):
            g0 = 2 * kk
            process(g0, 0)

            @pl.when(g0 + 1 < nch)
            def _():
                process(g0 + 1, 1)
            return carry
        lax.fori_loop(0, (nch + 1) // 2, pair, 0)

        plsc.subcore_barrier()
        pltpu.sync_copy(out_sh.at[pl.ds(base_r, RPT), :],
                        out_hbm.at[cc, pl.ds(base_r, RPT), :])
        pltpu.sync_copy(den_sh.at[pl.ds(base_r, RPT), :],
                        den_hbm.at[cc, pl.ds(base_r, RPT), :])

    return k(srcp, dstp, SD, LE, hmat)


def _fold(Wmat, a_vec):
    return jnp.einsum('dhf,hf->dh', Wmat.reshape(Wmat.shape[0], H, F), a_vec)


def kernel(x, edge_index, edge_attr, W1, a_src1, a_dst1, We1, a_e1,
           W2, a_src2, a_dst2, We2, a_e2):
    Bsd1 = jnp.concatenate([_fold(W1, a_src1), _fold(W1, a_dst1)], axis=1)
    Bsd2 = jnp.concatenate([_fold(W2, a_src2), _fold(W2, a_dst2)], axis=1)
    Be = jnp.concatenate([_fold(We1, a_e1), _fold(We2, a_e2)], axis=1)
    Ex = jnp.repeat(jnp.eye(16, dtype=jnp.float32)[:, :H], F, axis=1)

    BigBsd1 = jnp.kron(jnp.eye(8, dtype=jnp.float32), Bsd1)
    BigBsd2 = jnp.kron(jnp.eye(8, dtype=jnp.float32), Bsd2)
    BigBe = jnp.kron(jnp.eye(8, dtype=jnp.float32), Be)

    xp = jnp.zeros((NP, D), jnp.float32).at[:N].set(x)
    srcp = jnp.full((EP,), N, jnp.int32).at[:E].set(edge_index[0])
    dstp = jnp.full((EP,), N, jnp.int32).at[:E].set(edge_index[1])

    h1, SD1r = _h_sd(xp, W1, BigBsd1, 2048)
    SD1 = SD1r.reshape(NP, 16)
    LE = _le_mm(edge_attr, Be, 4000, EP // 8).reshape(EP, 16)
    P1, DEN1 = _edge_layer(srcp, dstp, SD1, LE, h1, le_hi=False)

    h2, SD2r = _mid(P1, DEN1, Ex, W2, BigBsd2)
    SD2 = SD2r.reshape(NP, 16)
    P2, DEN2 = _edge_layer(srcp, dstp, SD2, LE, h2, le_hi=True)

    out = _post(P2, DEN2, Ex)
    return out[:N][None]


# speedup vs baseline: 1.0200x; 1.0200x over previous
import functools

import jax
import jax.numpy as jnp
from jax import lax
from jax.experimental import pallas as pl
from jax.experimental.pallas import tpu as pltpu
from jax.experimental.pallas import tpu_sc as plsc

N = 10000
E = 320000
D = 128
DE = 16
H = 8
F = 16
ALPHA = 0.2

NP = 10240
NW = 32
C = 104
NCH0 = 119
NCH1 = 76
EPT0 = NCH0 * C
EPT1 = NCH1 * C
EP = 16 * (EPT0 + EPT1)
RPT = NP // 16


def _mm(a, b, blk, pack=1):
    M, K = a.shape
    Nc = b.shape[1]

    def body(a_ref, b_ref, o_ref):
        av = a_ref[...]
        if pack > 1:
            av = av.reshape(blk // pack, K * pack)
        o_ref[...] = jnp.dot(av, b_ref[...],
                             preferred_element_type=jnp.float32)

    return pl.pallas_call(
        body,
        grid=(M // blk,),
        in_specs=[pl.BlockSpec((blk, K), lambda i: (i, 0)),
                  pl.BlockSpec((K * pack, Nc), lambda i: (0, 0))],
        out_specs=pl.BlockSpec((blk // pack, Nc), lambda i: (i, 0)),
        out_shape=jax.ShapeDtypeStruct((M // pack, Nc), jnp.float32),
    )(a, b)


def _h_sd(xp, W, BigBsd, blk):
    def body(a_ref, w_ref, b_ref, h_ref, sd_ref):
        av = a_ref[...]
        h_ref[...] = jnp.dot(av, w_ref[...], preferred_element_type=jnp.float32)
        av8 = av.reshape(blk // 8, 8 * D)
        sd_ref[...] = jnp.dot(av8, b_ref[...],
                              preferred_element_type=jnp.float32)

    return pl.pallas_call(
        body,
        grid=(NP // blk,),
        in_specs=[pl.BlockSpec((blk, D), lambda i: (i, 0)),
                  pl.BlockSpec((D, D), lambda i: (0, 0)),
                  pl.BlockSpec((8 * D, D), lambda i: (0, 0))],
        out_specs=[pl.BlockSpec((blk, D), lambda i: (i, 0)),
                   pl.BlockSpec((blk // 8, D), lambda i: (i, 0))],
        out_shape=[jax.ShapeDtypeStruct((NP, D), jnp.float32),
                   jax.ShapeDtypeStruct((NP // 8, D), jnp.float32)],
    )(xp, W, BigBsd)


def _le_mm(ea, Be, blk, out_rows):
    E_ = ea.shape[0]
    ea3 = ea.reshape(E_ // 8, 8, 16)

    def body(a_ref, b_ref, o_ref):
        av = a_ref[...]
        le = lax.dot_general(av, b_ref[...], (((2,), (0,)), ((), ())),
                             preferred_element_type=jnp.float32)
        o_ref[...] = le.reshape(blk, 128)

    return pl.pallas_call(
        body,
        grid=(E_ // 8 // blk,),
        in_specs=[pl.BlockSpec((blk, 8, 16), lambda i: (i, 0, 0)),
                  pl.BlockSpec((16, 16), lambda i: (0, 0))],
        out_specs=pl.BlockSpec((blk, 128), lambda i: (i, 0)),
        out_shape=jax.ShapeDtypeStruct((out_rows, 128), jnp.float32),
    )(ea3, Be)


def _combine(p_ref, d_ref, ex_ref):
    o = p_ref[0] + p_ref[1]
    r = 1.0 / (d_ref[0] + d_ref[1] + 1e-16)
    rx = jnp.dot(r, ex_ref[...], preferred_element_type=jnp.float32)
    return o * rx


def _mid(P, DEN, Ex, W2, Bsd2):
    blk = 1024

    def body(p_ref, d_ref, ex_ref, w_ref, b_ref, h2_ref, sd_ref):
        o = _combine(p_ref, d_ref, ex_ref)
        hp = jnp.where(o > 0, o, jnp.exp(o) - 1.0)
        h2_ref[...] = jnp.dot(hp, w_ref[...], preferred_element_type=jnp.float32)
        hp8 = hp.reshape(blk // 8, 8 * D)
        sd_ref[...] = jnp.dot(hp8, b_ref[...], preferred_element_type=jnp.float32)

    return pl.pallas_call(
        body,
        grid=(NP // blk,),
        in_specs=[pl.BlockSpec((2, blk, D), lambda i: (0, i, 0)),
                  pl.BlockSpec((2, blk, 16), lambda i: (0, i, 0)),
                  pl.BlockSpec((16, D), lambda i: (0, 0)),
                  pl.BlockSpec((D, D), lambda i: (0, 0)),
                  pl.BlockSpec((8 * D, D), lambda i: (0, 0))],
        out_specs=[pl.BlockSpec((blk, D), lambda i: (i, 0)),
                   pl.BlockSpec((blk // 8, D), lambda i: (i, 0))],
        out_shape=[jax.ShapeDtypeStruct((NP, D), jnp.float32),
                   jax.ShapeDtypeStruct((NP // 8, D), jnp.float32)],
    )(P, DEN, Ex, W2, Bsd2)


def _post(P, DEN, Ex):
    blk = 1024

    def body(p_ref, d_ref, ex_ref, o_ref):
        o = _combine(p_ref, d_ref, ex_ref)
        acc = o[:, 0:16]
        for hh in range(1, H):
            acc = acc + o[:, hh * 16:(hh + 1) * 16]
        o_ref[...] = acc * (1.0 / H)

    return pl.pallas_call(
        body,
        grid=(NP // blk,),
        in_specs=[pl.BlockSpec((2, blk, D), lambda i: (0, i, 0)),
                  pl.BlockSpec((2, blk, 16), lambda i: (0, i, 0)),
                  pl.BlockSpec((16, D), lambda i: (0, 0))],
        out_specs=pl.BlockSpec((blk, 16), lambda i: (i, 0)),
        out_shape=jax.ShapeDtypeStruct((NP, 16), jnp.float32),
    )(P, DEN, Ex)


_DNUMS = lax.GatherDimensionNumbers(
    offset_dims=(), collapsed_slice_dims=(0,), start_index_map=(0,))


def _dg(v, idx):
    return lax.gather(v, idx[:, None], _DNUMS, (1,),
                      mode=lax.GatherScatterMode.PROMISE_IN_BOUNDS)


def _edge_layer(srcp, dstp, SD, LE, hmat, le_hi):
    mesh = plsc.VectorSubcoreMesh(core_axis_name="c", subcore_axis_name="s")

    @functools.partial(
        pl.kernel,
        out_type=(jax.ShapeDtypeStruct((2, NP, D), jnp.float32),
                  jax.ShapeDtypeStruct((2, NP, 16), jnp.float32)),
        mesh=mesh,
        compiler_params=pltpu.CompilerParams(use_tc_tiling_on_sc=False),
        scratch_types=[
            [pltpu.VMEM((C,), jnp.int32) for _ in range(2)],
            [pltpu.VMEM((C,), jnp.int32) for _ in range(2)],
            [pltpu.VMEM((C, 16), jnp.float32) for _ in range(2)],
            [pltpu.VMEM((C, 16), jnp.float32) for _ in range(2)],
            [pltpu.VMEM((C, 16), jnp.float32) for _ in range(2)],
            [pltpu.VMEM((C, D), jnp.float32) for _ in range(2)],
            pltpu.VMEM((C, 16), jnp.float32),
            pltpu.VMEM_SHARED((NP, D), jnp.float32),
            pltpu.VMEM_SHARED((NP, 16), jnp.float32),
            [pltpu.SemaphoreType.DMA for _ in range(2)],
            [pltpu.SemaphoreType.DMA for _ in range(2)],
        ],
    )
    def k(src_hbm, dst_hbm, sd_hbm, le_hbm, h_hbm, out_hbm, den_hbm,
          idxs, idxd, sds, sdd, leb, rows, exb, out_sh, den_sh, sems, semh):
        cc = lax.axis_index("c")
        ss = lax.axis_index("s")
        ii = lax.iota(jnp.int32, 16)
        shift8 = (ii & 7) + 8
        zero16 = jnp.zeros((16,), jnp.float32)
        base_r = ss * RPT
        ebase = jnp.where(cc == 0, ss * EPT0, 16 * EPT0 + ss * EPT1)
        nch = jnp.where(cc == 0, NCH0, NCH1)

        @plsc.parallel_loop(0, C, unroll=4)
        def zrow(i):
            exb[i, :] = zero16
            for j in range(D // 16):
                rows[0][i, pl.ds(j * 16, 16)] = zero16
        for kk in range(RPT // C):
            pltpu.sync_copy(exb, den_sh.at[pl.ds(base_r + kk * C, C), :])
            pltpu.sync_copy(rows[0], out_sh.at[pl.ds(base_r + kk * C, C), :])
        rem = RPT % C
        if rem:
            r0 = base_r + (RPT // C) * C
            pltpu.sync_copy(exb.at[pl.ds(0, rem), :],
                            den_sh.at[pl.ds(r0, rem), :])
            pltpu.sync_copy(rows[0].at[pl.ds(0, rem), :],
                            out_sh.at[pl.ds(r0, rem), :])
        plsc.subcore_barrier()

        def prep(g, b):
            base = ebase + g * C
            pltpu.sync_copy(src_hbm.at[pl.ds(base, C)], idxs[b])
            pltpu.sync_copy(dst_hbm.at[pl.ds(base, C)], idxd[b])
            pltpu.async_copy(sd_hbm.at[idxs[b]], sds[b], sems[b])
            pltpu.async_copy(sd_hbm.at[idxd[b]], sdd[b], sems[b])
            pltpu.async_copy(le_hbm.at[pl.ds(base, C), :], leb[b], sems[b])

        def process(g0, b):
            @pl.when(g0 + 1 < nch)
            def _():
                pltpu.async_copy(h_hbm.at[idxs[1 - b]], rows[1 - b],
                                 semh[1 - b])

            pltpu.make_async_copy(sd_hbm.at[idxs[b]], sds[b], sems[b]).wait()
            pltpu.make_async_copy(sd_hbm.at[idxd[b]], sdd[b], sems[b]).wait()
            pltpu.make_async_copy(le_hbm.at[pl.ds(0, C), :], leb[b],
                                  sems[b]).wait()

            @plsc.parallel_loop(0, C, unroll=4)
            def edge_ex(e):
                vs = sds[b][e, :]
                vd = _dg(sdd[b][e, :], shift8)
                if le_hi:
                    vl = _dg(leb[b][e, :], shift8)
                else:
                    vl = leb[b][e, :]
                z = vs + vd + vl
                z = jnp.where(z > 0, z, z * ALPHA)
                exb[e, :] = jnp.exp(z)

            pltpu.sync_copy(exb, den_sh.at[idxd[b]], add=True)
            pltpu.make_async_copy(h_hbm.at[idxs[b]], rows[b], semh[b]).wait()

            @plsc.parallel_loop(0, C, unroll=4)
            def edge_scale(e):
                a = exb[e, :]
                for hh in range(H):
                    fh = jnp.full((16,), hh, jnp.int32)
                    sc = _dg(a, fh)
                    rows[b][e, pl.ds(hh * 16, 16)] = (
                        rows[b][e, pl.ds(hh * 16, 16)] * sc)

            pltpu.sync_copy(rows[b], out_sh.at[idxd[b]], add=True)

            @pl.when(g0 + 2 < nch)
            def _():
                prep(g0 + 2, b)

        prep(0, 0)
        pltpu.async_copy(h_hbm.at[idxs[0]], rows[0], semh[0])
        prep(1, 1)

        def pair(kk, carry):
            g0 = 2 * kk
            process(g0, 0)

            @pl.when(g0 + 1 < nch)
            def _():
                process(g0 + 1, 1)
            return carry
        lax.fori_loop(0, (nch + 1) // 2, pair, 0)

        plsc.subcore_barrier()
        pltpu.sync_copy(out_sh.at[pl.ds(base_r, RPT), :],
                        out_hbm.at[cc, pl.ds(base_r, RPT), :])
        pltpu.sync_copy(den_sh.at[pl.ds(base_r, RPT), :],
                        den_hbm.at[cc, pl.ds(base_r, RPT), :])

    return k(srcp, dstp, SD, LE, hmat)


def _fold(Wmat, a_vec):
    return jnp.einsum('dhf,hf->dh', Wmat.reshape(Wmat.shape[0], H, F), a_vec)


def kernel(x, edge_index, edge_attr, W1, a_src1, a_dst1, We1, a_e1,
           W2, a_src2, a_dst2, We2, a_e2):
    Bsd1 = jnp.concatenate([_fold(W1, a_src1), _fold(W1, a_dst1)], axis=1)
    Bsd2 = jnp.concatenate([_fold(W2, a_src2), _fold(W2, a_dst2)], axis=1)
    Be = jnp.concatenate([_fold(We1, a_e1), _fold(We2, a_e2)], axis=1)
    Ex = jnp.repeat(jnp.eye(16, dtype=jnp.float32)[:, :H], F, axis=1)

    BigBsd1 = jnp.kron(jnp.eye(8, dtype=jnp.float32), Bsd1)
    BigBsd2 = jnp.kron(jnp.eye(8, dtype=jnp.float32), Bsd2)
    BigBe = jnp.kron(jnp.eye(8, dtype=jnp.float32), Be)

    xp = jnp.zeros((NP, D), jnp.float32).at[:N].set(x)
    srcp = jnp.full((EP,), N, jnp.int32).at[:E].set(edge_index[0])
    dstp = jnp.full((EP,), N, jnp.int32).at[:E].set(edge_index[1])

    h1, SD1r = _h_sd(xp, W1, BigBsd1, 2048)
    SD1 = SD1r.reshape(NP, 16)
    LE = _le_mm(edge_attr, Be, 4000, EP // 8).reshape(EP, 16)
    P1, DEN1 = _edge_layer(srcp, dstp, SD1, LE, h1, le_hi=False)

    h2, SD2r = _mid(P1, DEN1, Ex, W2, BigBsd2)
    SD2 = SD2r.reshape(NP, 16)
    P2, DEN2 = _edge_layer(srcp, dstp, SD2, LE, h2, le_hi=True)

    out = _post(P2, DEN2, Ex)
    return out[:N][None]


# speedup vs baseline: 1.1265x; 1.1044x over previous
import functools

import jax
import jax.numpy as jnp
from jax import lax
from jax.experimental import pallas as pl
from jax.experimental.pallas import tpu as pltpu
from jax.experimental.pallas import tpu_sc as plsc

N = 10000
E = 320000
D = 128
DE = 16
H = 8
F = 16
ALPHA = 0.2

NP = 10240
NW = 32
C = 104
NCH0 = 119
NCH1 = 76
EPT0 = NCH0 * C
EPT1 = NCH1 * C
EP = 16 * (EPT0 + EPT1)
RPT = NP // 16


def _mm(a, b, blk, pack=1):
    M, K = a.shape
    Nc = b.shape[1]

    def body(a_ref, b_ref, o_ref):
        av = a_ref[...]
        if pack > 1:
            av = av.reshape(blk // pack, K * pack)
        o_ref[...] = jnp.dot(av, b_ref[...],
                             preferred_element_type=jnp.float32)

    return pl.pallas_call(
        body,
        grid=(M // blk,),
        in_specs=[pl.BlockSpec((blk, K), lambda i: (i, 0)),
                  pl.BlockSpec((K * pack, Nc), lambda i: (0, 0))],
        out_specs=pl.BlockSpec((blk // pack, Nc), lambda i: (i, 0)),
        out_shape=jax.ShapeDtypeStruct((M // pack, Nc), jnp.float32),
    )(a, b)


def _h_sd(xp, W, BigBsd, blk):
    def body(a_ref, w_ref, b_ref, h_ref, sd_ref):
        av = a_ref[...]
        h_ref[...] = jnp.dot(av, w_ref[...], preferred_element_type=jnp.float32)
        av8 = av.reshape(blk // 8, 8 * D)
        sd_ref[...] = jnp.dot(av8, b_ref[...],
                              preferred_element_type=jnp.float32)

    return pl.pallas_call(
        body,
        grid=(NP // blk,),
        in_specs=[pl.BlockSpec((blk, D), lambda i: (i, 0)),
                  pl.BlockSpec((D, D), lambda i: (0, 0)),
                  pl.BlockSpec((8 * D, D), lambda i: (0, 0))],
        out_specs=[pl.BlockSpec((blk, D), lambda i: (i, 0)),
                   pl.BlockSpec((blk // 8, D), lambda i: (i, 0))],
        out_shape=[jax.ShapeDtypeStruct((NP, D), jnp.float32),
                   jax.ShapeDtypeStruct((NP // 8, D), jnp.float32)],
    )(xp, W, BigBsd)


def _le_mm(ea, Be, blk, out_rows):
    E_ = ea.shape[0]
    ea3 = ea.reshape(E_ // 8, 8, 16)

    def body(a_ref, b_ref, o_ref):
        av = a_ref[...]
        le = lax.dot_general(av, b_ref[...], (((2,), (0,)), ((), ())),
                             preferred_element_type=jnp.float32)
        o_ref[...] = le.reshape(blk, 128)

    return pl.pallas_call(
        body,
        grid=(E_ // 8 // blk,),
        in_specs=[pl.BlockSpec((blk, 8, 16), lambda i: (i, 0, 0)),
                  pl.BlockSpec((16, 16), lambda i: (0, 0))],
        out_specs=pl.BlockSpec((blk, 128), lambda i: (i, 0)),
        out_shape=jax.ShapeDtypeStruct((out_rows, 128), jnp.float32),
    )(ea3, Be)


def _combine(p_ref, d_ref, ex_ref):
    o = p_ref[0] + p_ref[1]
    r = 1.0 / (d_ref[0] + d_ref[1] + 1e-16)
    rx = jnp.dot(r, ex_ref[...], preferred_element_type=jnp.float32)
    return o * rx


def _mid(P, DEN, Ex, W2, Bsd2):
    blk = 1024

    def body(p_ref, d_ref, ex_ref, w_ref, b_ref, h2_ref, sd_ref):
        o = _combine(p_ref, d_ref, ex_ref)
        hp = jnp.where(o > 0, o, jnp.exp(o) - 1.0)
        h2_ref[...] = jnp.dot(hp, w_ref[...], preferred_element_type=jnp.float32)
        hp8 = hp.reshape(blk // 8, 8 * D)
        sd_ref[...] = jnp.dot(hp8, b_ref[...], preferred_element_type=jnp.float32)

    return pl.pallas_call(
        body,
        grid=(NP // blk,),
        in_specs=[pl.BlockSpec((2, blk, D), lambda i: (0, i, 0)),
                  pl.BlockSpec((2, blk, 16), lambda i: (0, i, 0)),
                  pl.BlockSpec((16, D), lambda i: (0, 0)),
                  pl.BlockSpec((D, D), lambda i: (0, 0)),
                  pl.BlockSpec((8 * D, D), lambda i: (0, 0))],
        out_specs=[pl.BlockSpec((blk, D), lambda i: (i, 0)),
                   pl.BlockSpec((blk // 8, D), lambda i: (i, 0))],
        out_shape=[jax.ShapeDtypeStruct((NP, D), jnp.float32),
                   jax.ShapeDtypeStruct((NP // 8, D), jnp.float32)],
    )(P, DEN, Ex, W2, Bsd2)


def _post(P, DEN, Ex):
    blk = 1024

    def body(p_ref, d_ref, ex_ref, o_ref):
        o = _combine(p_ref, d_ref, ex_ref)
        acc = o[:, 0:16]
        for hh in range(1, H):
            acc = acc + o[:, hh * 16:(hh + 1) * 16]
        o_ref[...] = acc * (1.0 / H)

    return pl.pallas_call(
        body,
        grid=(NP // blk,),
        in_specs=[pl.BlockSpec((2, blk, D), lambda i: (0, i, 0)),
                  pl.BlockSpec((2, blk, 16), lambda i: (0, i, 0)),
                  pl.BlockSpec((16, D), lambda i: (0, 0))],
        out_specs=pl.BlockSpec((blk, 16), lambda i: (i, 0)),
        out_shape=jax.ShapeDtypeStruct((NP, 16), jnp.float32),
    )(P, DEN, Ex)


_DNUMS = lax.GatherDimensionNumbers(
    offset_dims=(), collapsed_slice_dims=(0,), start_index_map=(0,))


def _dg(v, idx):
    return lax.gather(v, idx[:, None], _DNUMS, (1,),
                      mode=lax.GatherScatterMode.PROMISE_IN_BOUNDS)


def _edge_layer(srcp, dstp, SD, LE, hmat, le_hi):
    mesh = plsc.VectorSubcoreMesh(core_axis_name="c", subcore_axis_name="s")

    @functools.partial(
        pl.kernel,
        out_type=(jax.ShapeDtypeStruct((2, NP, D), jnp.float32),
                  jax.ShapeDtypeStruct((2, NP, 16), jnp.float32)),
        mesh=mesh,
        compiler_params=pltpu.CompilerParams(use_tc_tiling_on_sc=False),
        scratch_types=[
            [pltpu.VMEM((C,), jnp.int32) for _ in range(2)],
            [pltpu.VMEM((C,), jnp.int32) for _ in range(2)],
            [pltpu.VMEM((C, 16), jnp.float32) for _ in range(2)],
            [pltpu.VMEM((C, 16), jnp.float32) for _ in range(2)],
            [pltpu.VMEM((C, 16), jnp.float32) for _ in range(2)],
            [pltpu.VMEM((C, D), jnp.float32) for _ in range(2)],
            pltpu.VMEM((C, 16), jnp.float32),
            pltpu.VMEM_SHARED((NP, D), jnp.float32),
            pltpu.VMEM_SHARED((NP, 16), jnp.float32),
            [pltpu.SemaphoreType.DMA for _ in range(2)],
            [pltpu.SemaphoreType.DMA for _ in range(2)],
        ],
    )
    def k(src_hbm, dst_hbm, sd_hbm, le_hbm, h_hbm, out_hbm, den_hbm,
          idxs, idxd, sds, sdd, leb, rows, exb, out_sh, den_sh, sems, semh):
        cc = lax.axis_index("c")
        ss = lax.axis_index("s")
        ii = lax.iota(jnp.int32, 16)
        shift8 = (ii & 7) + 8
        zero16 = jnp.zeros((16,), jnp.float32)
        base_r = ss * RPT
        ebase = jnp.where(cc == 0, ss * EPT0, 16 * EPT0 + ss * EPT1)
        nch = jnp.where(cc == 0, NCH0, NCH1)

        @plsc.parallel_loop(0, C, unroll=4)
        def zrow(i):
            exb[i, :] = zero16
            for j in range(D // 16):
                rows[0][i, pl.ds(j * 16, 16)] = zero16
        for kk in range(RPT // C):
            pltpu.sync_copy(exb, den_sh.at[pl.ds(base_r + kk * C, C), :])
            pltpu.sync_copy(rows[0], out_sh.at[pl.ds(base_r + kk * C, C), :])
        rem = RPT % C
        if rem:
            r0 = base_r + (RPT // C) * C
            pltpu.sync_copy(exb.at[pl.ds(0, rem), :],
                            den_sh.at[pl.ds(r0, rem), :])
            pltpu.sync_copy(rows[0].at[pl.ds(0, rem), :],
                            out_sh.at[pl.ds(r0, rem), :])
        plsc.subcore_barrier()

        def prep(g, b):
            base = ebase + g * C
            pltpu.sync_copy(src_hbm.at[pl.ds(base, C)], idxs[b])
            pltpu.sync_copy(dst_hbm.at[pl.ds(base, C)], idxd[b])
            pltpu.async_copy(sd_hbm.at[idxs[b]], sds[b], sems[b])
            pltpu.async_copy(sd_hbm.at[idxd[b]], sdd[b], sems[b])
            pltpu.async_copy(le_hbm.at[pl.ds(base, C), :], leb[b], sems[b])

        def process(g0, b):
            @pl.when(g0 + 1 < nch)
            def _():
                pltpu.async_copy(h_hbm.at[idxs[1 - b]], rows[1 - b],
                                 semh[1 - b])

            pltpu.make_async_copy(sd_hbm.at[idxs[b]], sds[b], sems[b]).wait()
            pltpu.make_async_copy(sd_hbm.at[idxd[b]], sdd[b], sems[b]).wait()
            pltpu.make_async_copy(le_hbm.at[pl.ds(0, C), :], leb[b],
                                  sems[b]).wait()

            @plsc.parallel_loop(0, C, unroll=4)
            def edge_ex(e):
                vs = sds[b][e, :]
                vd = _dg(sdd[b][e, :], shift8)
                if le_hi:
                    vl = _dg(leb[b][e, :], shift8)
                else:
                    vl = leb[b][e, :]
                z = vs + vd + vl
                z = jnp.where(z > 0, z, z * ALPHA)
                exb[e, :] = jnp.exp(z)

            pltpu.sync_copy(exb, den_sh.at[idxd[b]], add=True)
            pltpu.make_async_copy(h_hbm.at[idxs[b]], rows[b], semh[b]).wait()

            @plsc.parallel_loop(0, C, unroll=4)
            def edge_scale(e):
                a = exb[e, :]
                for hh in range(H):
                    fh = jnp.full((16,), hh, jnp.int32)
                    sc = _dg(a, fh)
                    rows[b][e, pl.ds(hh * 16, 16)] = (
                        rows[b][e, pl.ds(hh * 16, 16)] * sc)

            pltpu.sync_copy(rows[b], out_sh.at[idxd[b]], add=True)

            @pl.when(g0 + 2 < nch)
            def _():
                prep(g0 + 2, b)

        prep(0, 0)
        pltpu.async_copy(h_hbm.at[idxs[0]], rows[0], semh[0])
        prep(1, 1)

        def pair(kk, carry):
            g0 = 2 * kk
            process(g0, 0)

            @pl.when(g0 + 1 < nch)
            def _():
                process(g0 + 1, 1)
            return carry
        lax.fori_loop(0, (nch + 1) // 2, pair, 0)

        plsc.subcore_barrier()
        pltpu.sync_copy(out_sh.at[pl.ds(base_r, RPT), :],
                        out_hbm.at[cc, pl.ds(base_r, RPT), :])
        pltpu.sync_copy(den_sh.at[pl.ds(base_r, RPT), :],
                        den_hbm.at[cc, pl.ds(base_r, RPT), :])

    return k(srcp, dstp, SD, LE, hmat)


def _fold(Wmat, a_vec):
    return jnp.einsum('dhf,hf->dh', Wmat.reshape(Wmat.shape[0], H, F), a_vec)


def kernel(x, edge_index, edge_attr, W1, a_src1, a_dst1, We1, a_e1,
           W2, a_src2, a_dst2, We2, a_e2):
    Bsd1 = jnp.concatenate([_fold(W1, a_src1), _fold(W1, a_dst1)], axis=1)
    Bsd2 = jnp.concatenate([_fold(W2, a_src2), _fold(W2, a_dst2)], axis=1)
    Be = jnp.concatenate([_fold(We1, a_e1), _fold(We2, a_e2)], axis=1)
    Ex = jnp.repeat(jnp.eye(16, dtype=jnp.float32)[:, :H], F, axis=1)

    BigBsd1 = jnp.kron(jnp.eye(8, dtype=jnp.float32), Bsd1)
    BigBsd2 = jnp.kron(jnp.eye(8, dtype=jnp.float32), Bsd2)
    BigBe = jnp.kron(jnp.eye(8, dtype=jnp.float32), Be)

    xp = jnp.zeros((NP, D), jnp.float32).at[:N].set(x)
    srcp = jnp.full((EP,), N, jnp.int32).at[:E].set(edge_index[0])
    dstp = jnp.full((EP,), N, jnp.int32).at[:E].set(edge_index[1])

    h1 = _mm(xp, W1, 2048)
    SD1 = _mm(xp, BigBsd1, 2048, pack=8).reshape(NP, 16)
    LE = _le_mm(edge_attr, Be, 4000, EP // 8).reshape(EP, 16)
    P1, DEN1 = _edge_layer(srcp, dstp, SD1, LE, h1, le_hi=False)

    h2, SD2r = _mid(P1, DEN1, Ex, W2, BigBsd2)
    SD2 = SD2r.reshape(NP, 16)
    P2, DEN2 = _edge_layer(srcp, dstp, SD2, LE, h2, le_hi=True)

    out = _post(P2, DEN2, Ex)
    return out[:N][None]
